# baseline (speedup 1.0000x reference)
import functools

import jax
import jax.numpy as jnp
from jax import lax
from jax.experimental import pallas as pl
from jax.experimental.pallas import tpu as pltpu
from jax.experimental.pallas import tpu_sc as plsc

D = 64
K = 128
SB = 6
NRB = 3
SBE = SB * K
NS = 16
NC = 2
EDGE_ALIGN = NS * K * SB
NPAD_BIG = 50176
NPAD_SMALL = 8192
APAD = 128

_GDN = lax.GatherDimensionNumbers(
    offset_dims=(), collapsed_slice_dims=(0,), start_index_map=(0,))


def _lane_bcast(v16, e):
    idx = jnp.full((16, 1), e, jnp.int32)
    return lax.gather(v16, idx, _GDN, slice_sizes=(1,),
                      mode=lax.GatherScatterMode.PROMISE_IN_BOUNDS)


def _pair_body(EA, EB, NOUT, skipB, rows_max_b,
               idxA, valA, xA, idxB, valB, xB, out,
               idx_st, val_st, colc, lrowc, valc, rb, lr, accum, gsem, ssem):
    nhalf = NOUT // 2
    cid = lax.axis_index("c")
    sid = lax.axis_index("s")
    base = cid * nhalf

    zeros16 = jnp.zeros((16,), jnp.float32)

    def zrow(i, _):
        for j in range(4):
            rb[0][i, pl.ds(j * 16, 16)] = zeros16
        return 0

    lax.fori_loop(0, K, zrow, 0)
    acc_rows_per_sub = (nhalf + APAD) // NS
    z0 = sid * acc_rows_per_sub
    zfull, zrem = acc_rows_per_sub // K, acc_rows_per_sub % K
    for i in range(zfull):
        pltpu.sync_copy(rb[0], accum.at[pl.ds(z0 + i * K, K)])
    if zrem:
        pltpu.sync_copy(rb[0].at[pl.ds(0, zrem)],
                        accum.at[pl.ds(z0 + zfull * K, zrem)])
    plsc.subcore_barrier()

    zeros16i = jnp.zeros((16,), jnp.int32)
    trash16 = jnp.full((16,), nhalf, jnp.int32) + sid

    def process(idx_hbm, val_hbm, x_hbm, nchunks):
        nsb = nchunks // SB
        e0 = sid * (nchunks * K)

        def stage(s):
            off = e0 + s * SBE
            pltpu.sync_copy(idx_hbm.at[:, pl.ds(off, SBE)], idx_st)
            pltpu.sync_copy(val_hbm.at[pl.ds(off, SBE)], val_st)

        def g_start(i):
            b = i % NRB
            pltpu.async_copy(x_hbm.at[colc.at[pl.ds(i * K, K)]],
                             rb[b], gsem[b])

        def g_wait(i):
            b = i % NRB
            pltpu.make_async_copy(x_hbm.at[colc.at[pl.ds(i * K, K)]],
                                  rb[b], gsem[b]).wait()

        def s_start(i):
            b = i % NRB
            pltpu.async_copy(rb[b], accum.at[lr[b]], ssem[b], add=True)

        def s_wait(b):
            pltpu.make_async_copy(rb[b], accum.at[lr[b]], ssem[b]).wait()

        def fgroup(g, cnt):
            coff = g * 16
            r16 = idx_st[0, pl.ds(coff, 16)]
            c16 = idx_st[1, pl.ds(coff, 16)]
            v16 = val_st[pl.ds(coff, 16)]
            lrl = r16 - base
            ok = (lrl >= 0) & (lrl < nhalf)
            plsc.store_compressed(colc.at[pl.ds(cnt, 16)], c16, mask=ok)
            plsc.store_compressed(lrowc.at[pl.ds(cnt, 16)], lrl, mask=ok)
            plsc.store_compressed(valc.at[pl.ds(cnt, 16)], v16, mask=ok)
            pc = plsc.all_reduce_population_count(ok)
            return cnt + lax.reduce_max(pc, (0,))

        def scale(i):
            b = i % NRB
            rbuf, lrow = rb[b], lr[b]

            def group(q, _):
                off = i * K + q * 16
                lrow[pl.ds(q * 16, 16)] = lrowc[pl.ds(off, 16)]
                v16 = valc[pl.ds(off, 16)]
                for e in range(16):
                    vb = _lane_bcast(v16, e)
                    ei = q * 16 + e
                    for j in range(4):
                        rbuf[ei, pl.ds(j * 16, 16)] = (
                            rbuf[ei, pl.ds(j * 16, 16)] * vb)
                return 0

            lax.fori_loop(0, K // 16, group, 0)

        def gblock_start(i, nblk):
            @pl.when(i < nblk)
            def _():
                if i >= NRB:
                    s_wait(i % NRB)
                g_start(i)

        def gblock_finish(i, nblk):
            @pl.when(i < nblk)
            def _():
                g_wait(i)
                scale(i)
                s_start(i)

        def sb_body(s, nblk_prev):
            stage(s)
            for b in range(NRB):
                @pl.when(b < nblk_prev)
                def _(b=b):
                    s_wait(b)
            cnt = lax.fori_loop(0, SBE // 16, fgroup, jnp.int32(0))
            for p in range(8):
                colc[pl.ds(cnt + p * 16, 16)] = zeros16i
                lrowc[pl.ds(cnt + p * 16, 16)] = trash16
                valc[pl.ds(cnt + p * 16, 16)] = jnp.zeros((16,), jnp.float32)
            nblk = (cnt + K - 1) // K
            for i in range(SB + 1):
                if i < SB:
                    gblock_start(i, nblk)
                if i >= 1:
                    gblock_finish(i - 1, nblk)
            return nblk

        last_nblk = lax.fori_loop(0, nsb, sb_body, jnp.int32(0))
        for b in range(NRB):
            @pl.when(b < last_nblk)
            def _(b=b):
                s_wait(b)

    process(idxA, valA, xA, EA // (NS * K))
    if skipB:

        @pl.when(cid == 0)
        def _():
            process(idxB, valB, xB, EB // (NS * K))
    else:
        process(idxB, valB, xB, EB // (NS * K))
    plsc.subcore_barrier()

    wb_rows = nhalf // NS
    w0 = sid * wb_rows
    wfull, wrem = wb_rows // K, wb_rows % K
    for i in range(wfull):
        pltpu.sync_copy(accum.at[pl.ds(w0 + i * K, K)],
                        out.at[pl.ds(base + w0 + i * K, K)])
    if wrem:
        pltpu.sync_copy(accum.at[pl.ds(w0 + wfull * K, wrem)],
                        out.at[pl.ds(base + w0 + wfull * K, wrem)])


@functools.lru_cache(maxsize=None)
def _make_pair(EA, EB, NOUT, skipB):
    mesh = plsc.VectorSubcoreMesh(core_axis_name="c", subcore_axis_name="s")
    nhalf = NOUT // 2
    return pl.kernel(
        functools.partial(_pair_body, EA, EB, NOUT, skipB, None),
        out_type=jax.ShapeDtypeStruct((NOUT, D), jnp.float32),
        mesh=mesh,
        compiler_params=pltpu.CompilerParams(use_tc_tiling_on_sc=False,
                                             needs_layout_passes=False),
        scratch_types=dict(
            idx_st=pltpu.VMEM((2, SBE), jnp.int32),
            val_st=pltpu.VMEM((SBE,), jnp.float32),
            colc=pltpu.VMEM((SBE + 128,), jnp.int32),
            lrowc=pltpu.VMEM((SBE + 128,), jnp.int32),
            valc=pltpu.VMEM((SBE + 128,), jnp.float32),
            rb=[pltpu.VMEM((K, D), jnp.float32) for _ in range(NRB)],
            lr=[pltpu.VMEM((K,), jnp.int32) for _ in range(NRB)],
            accum=pltpu.VMEM_SHARED((nhalf + APAD, D), jnp.float32),
            gsem=[pltpu.SemaphoreType.DMA for _ in range(NRB)],
            ssem=[pltpu.SemaphoreType.DMA for _ in range(NRB)],
        ),
    )


def _spmm_pair(edA, xA, edB, xB, nout, skipB=False):
    idxA, vA = edA
    idxB, vB = edB
    k = _make_pair(idxA.shape[1], idxB.shape[1], nout, skipB)
    return k(idxA, vA, xA, idxB, vB, xB)


def _mm_body(x_ref, w_ref, b_ref, o_ref):
    o_ref[...] = jnp.dot(x_ref[...], w_ref[...],
                         preferred_element_type=jnp.float32,
                         precision=lax.Precision.HIGHEST) + b_ref[...]


def _proj(x, w, b):
    n = x.shape[0]
    bm = 1024 if n % 1024 == 0 else 512
    return pl.pallas_call(
        _mm_body,
        grid=(n // bm,),
        in_specs=[pl.BlockSpec((bm, D), lambda i: (i, 0)),
                  pl.BlockSpec((D, D), lambda i: (0, 0)),
                  pl.BlockSpec((1, D), lambda i: (0, 0))],
        out_specs=pl.BlockSpec((bm, D), lambda i: (i, 0)),
        out_shape=jax.ShapeDtypeStruct((n, D), jnp.float32),
    )(x, w, b.reshape(1, D))


def _mean3_body(a_ref, b_ref, c_ref, o_ref):
    o_ref[...] = (a_ref[...] + b_ref[...] + c_ref[...]) * (1.0 / 3.0)


def _mean3(a, b, c):
    n = a.shape[0]
    bm = 1024 if n % 1024 == 0 else 512
    spec = pl.BlockSpec((bm, D), lambda i: (i, 0))
    return pl.pallas_call(
        _mean3_body,
        grid=(n // bm,),
        in_specs=[spec, spec, spec],
        out_specs=spec,
        out_shape=jax.ShapeDtypeStruct((n, D), jnp.float32),
    )(a, b, c)


def _prep_edges(idx, val):
    e = idx.shape[1]
    epad = -(-e // EDGE_ALIGN) * EDGE_ALIGN
    return jnp.pad(idx, ((0, 0), (0, epad - e))), jnp.pad(val, (0, epad - e))


def kernel(hg_ul_idx, hg_ul_val, hg_ut_idx, hg_ut_val, hg_ua_idx, hg_ua_val,
           hg_l_idx, hg_l_val, hg_t_idx, hg_t_val, hg_a_idx, hg_a_val,
           vtoe_lu_idx, vtoe_lu_val, vtoe_tu_idx, vtoe_tu_val,
           vtoe_au_idx, vtoe_au_val, vtoe_ul_idx, vtoe_ul_val,
           vtoe_ut_idx, vtoe_ut_val, vtoe_ua_idx, vtoe_ua_val,
           user_emb, loc_emb, time_emb, act_emb, W_l, b_l, W_t, b_t, W_a, b_a):
    U, L = user_emb.shape[0], loc_emb.shape[0]

    hg_ul = _prep_edges(hg_ul_idx, hg_ul_val)
    hg_ut = _prep_edges(hg_ut_idx, hg_ut_val)
    hg_ua = _prep_edges(hg_ua_idx, hg_ua_val)
    hg_l = _prep_edges(hg_l_idx, hg_l_val)
    hg_t = _prep_edges(hg_t_idx, hg_t_val)
    hg_a = _prep_edges(hg_a_idx, hg_a_val)
    vt_lu = _prep_edges(vtoe_lu_idx, vtoe_lu_val)
    vt_tu = _prep_edges(vtoe_tu_idx, vtoe_tu_val)
    vt_au = _prep_edges(vtoe_au_idx, vtoe_au_val)
    vt_ul = _prep_edges(vtoe_ul_idx, vtoe_ul_val)
    vt_ut = _prep_edges(vtoe_ut_idx, vtoe_ut_val)
    vt_ua = _prep_edges(vtoe_ua_idx, vtoe_ua_val)

    pad_rows = lambda x: jnp.pad(x, ((0, NPAD_BIG - x.shape[0]), (0, 0)))
    ue = pad_rows(user_emb)
    u_l = _proj(ue, W_l, b_l)
    u_t = _proj(ue, W_t, b_t)
    u_a = _proj(ue, W_a, b_a)
    l = pad_rows(loc_emb)
    t, a = time_emb, act_emb

    u_l_all, u_t_all, u_a_all = [u_l], [u_t], [u_a]
    l_all, t_all, a_all = [l], [t], [a]
    for _ in range(2):
        u_l1 = _spmm_pair(hg_ul, u_l, vt_lu, l, NPAD_BIG)
        l1 = _spmm_pair(hg_l, l, vt_ul, u_l, NPAD_BIG)
        u_t1 = _spmm_pair(hg_ut, u_t, vt_tu, t, NPAD_BIG, skipB=True)
        t1 = _spmm_pair(hg_t, t, vt_ut, u_t, NPAD_SMALL)
        u_a1 = _spmm_pair(hg_ua, u_a, vt_au, a, NPAD_BIG, skipB=True)
        a1 = _spmm_pair(hg_a, a, vt_ua, u_a, NPAD_SMALL)
        u_l, l, u_t, t, u_a, a = u_l1, l1, u_t1, t1, u_a1, a1
        u_l_all.append(u_l); u_t_all.append(u_t); u_a_all.append(u_a)
        l_all.append(l); t_all.append(t); a_all.append(a)

    u_l_f = _mean3(*u_l_all)[:U]
    u_t_f = _mean3(*u_t_all)[:U]
    u_a_f = _mean3(*u_a_all)[:U]
    l_f = _mean3(*l_all)[:L]
    t_f = _mean3(*t_all)
    a_f = _mean3(*a_all)
    return (u_l_f, u_t_f, u_a_f, l_f, t_f, a_f)


# speedup vs baseline: 1.9817x; 1.9817x over previous
import functools

import jax
import jax.numpy as jnp
from jax import lax
from jax.experimental import pallas as pl
from jax.experimental.pallas import tpu as pltpu
from jax.experimental.pallas import tpu_sc as plsc

D = 64
K = 128
SB = 6
NRB = 3
SBE = SB * K
NS = 16
NC = 2
EDGE_ALIGN = NS * K * SB
NPAD_BIG = 50176
NPAD_SMALL = 8192
APAD = 256

_GDN = lax.GatherDimensionNumbers(
    offset_dims=(), collapsed_slice_dims=(0,), start_index_map=(0,))


def _lane_bcast(v16, e):
    idx = jnp.full((16, 1), e, jnp.int32)
    return lax.gather(v16, idx, _GDN, slice_sizes=(1,),
                      mode=lax.GatherScatterMode.PROMISE_IN_BOUNDS)


def _pair_body(EA, EB, NOUT, skipB, rows_max_b,
               idxA, valA, xA, idxB, valB, xB, out,
               idx_st, val_st, rb, lr, accum, gsem, ssem):
    nhalf = NOUT // 2
    cid = lax.axis_index("c")
    sid = lax.axis_index("s")
    base = cid * nhalf

    zeros16 = jnp.zeros((16,), jnp.float32)

    def zrow(i, _):
        for j in range(4):
            rb[0][i, pl.ds(j * 16, 16)] = zeros16
        return 0

    lax.fori_loop(0, K, zrow, 0)
    acc_rows_per_sub = (nhalf + APAD) // NS
    z0 = sid * acc_rows_per_sub
    zfull, zrem = acc_rows_per_sub // K, acc_rows_per_sub % K
    for i in range(zfull):
        pltpu.sync_copy(rb[0], accum.at[pl.ds(z0 + i * K, K)])
    if zrem:
        pltpu.sync_copy(rb[0].at[pl.ds(0, zrem)],
                        accum.at[pl.ds(z0 + zfull * K, zrem)])
    plsc.subcore_barrier()

    def process(idx_hbm, val_hbm, x_hbm, nchunks):
        nsb = nchunks // SB
        e0 = sid * (nchunks * K)

        def stage(s):
            off = e0 + s * SBE
            pltpu.sync_copy(idx_hbm.at[:, pl.ds(off, SBE)], idx_st)
            pltpu.sync_copy(val_hbm.at[pl.ds(off, SBE)], val_st)

        def g_start(cc):
            b = cc % NRB
            h = K // 2
            pltpu.async_copy(x_hbm.at[idx_st.at[1, pl.ds(cc * K, h)]],
                             rb[b].at[pl.ds(0, h)], gsem[b])
            pltpu.async_copy(x_hbm.at[idx_st.at[1, pl.ds(cc * K + h, h)]],
                             rb[b].at[pl.ds(h, h)], gsem[b])

        def g_wait(cc):
            b = cc % NRB
            h = K // 2
            pltpu.make_async_copy(x_hbm.at[idx_st.at[1, pl.ds(cc * K, h)]],
                                  rb[b].at[pl.ds(0, h)], gsem[b]).wait()
            pltpu.make_async_copy(x_hbm.at[idx_st.at[1, pl.ds(cc * K + h, h)]],
                                  rb[b].at[pl.ds(h, h)], gsem[b]).wait()

        def s_start(cc):
            b = cc % NRB
            pltpu.async_copy(rb[b], accum.at[lr[b]], ssem[b], add=True)

        def s_wait(b):
            pltpu.make_async_copy(rb[b], accum.at[lr[b]], ssem[b]).wait()

        def compute(cc):
            b = cc % NRB
            rbuf, lrow = rb[b], lr[b]

            def group(q, _):
                coff = cc * K + q * 16
                r16 = idx_st[0, pl.ds(coff, 16)]
                lrl = r16 - base
                ok = (lrl >= 0) & (lrl < nhalf)
                lrow[pl.ds(q * 16, 16)] = jnp.where(ok, lrl, nhalf + sid)
                v16 = val_st[pl.ds(coff, 16)]
                for e in range(16):
                    vb = _lane_bcast(v16, e)
                    ei = q * 16 + e
                    for j in range(4):
                        rbuf[ei, pl.ds(j * 16, 16)] = (
                            rbuf[ei, pl.ds(j * 16, 16)] * vb)
                return 0

            lax.fori_loop(0, K // 16, group, 0)

        def run_sb(first):
            if not first:
                s_wait(0)
            g_start(0)
            for cc in range(SB):
                if cc + 1 < SB:
                    if (not first) or cc >= 2:
                        s_wait((cc + 1) % NRB)
                    g_start(cc + 1)
                g_wait(cc)
                compute(cc)
                s_start(cc)

        stage(0)
        run_sb(True)

        def sb_body(s, _):
            stage(s)
            run_sb(False)
            return 0

        if nsb > 1:
            lax.fori_loop(1, nsb, sb_body, 0)
        for b in range(NRB):
            s_wait(b)

    process(idxA, valA, xA, EA // (NS * K))
    if skipB:

        @pl.when(cid == 0)
        def _():
            process(idxB, valB, xB, EB // (NS * K))
    else:
        process(idxB, valB, xB, EB // (NS * K))
    plsc.subcore_barrier()

    wb_rows = nhalf // NS
    w0 = sid * wb_rows
    wfull, wrem = wb_rows // K, wb_rows % K
    for i in range(wfull):
        pltpu.sync_copy(accum.at[pl.ds(w0 + i * K, K)],
                        out.at[pl.ds(base + w0 + i * K, K)])
    if wrem:
        pltpu.sync_copy(accum.at[pl.ds(w0 + wfull * K, wrem)],
                        out.at[pl.ds(base + w0 + wfull * K, wrem)])


@functools.lru_cache(maxsize=None)
def _make_pair(EA, EB, NOUT, skipB):
    mesh = plsc.VectorSubcoreMesh(core_axis_name="c", subcore_axis_name="s")
    nhalf = NOUT // 2
    return pl.kernel(
        functools.partial(_pair_body, EA, EB, NOUT, skipB, None),
        out_type=jax.ShapeDtypeStruct((NOUT, D), jnp.float32),
        mesh=mesh,
        compiler_params=pltpu.CompilerParams(use_tc_tiling_on_sc=False,
                                             needs_layout_passes=False),
        scratch_types=dict(
            idx_st=pltpu.VMEM((2, SBE), jnp.int32),
            val_st=pltpu.VMEM((SBE,), jnp.float32),
            rb=[pltpu.VMEM((K, D), jnp.float32) for _ in range(NRB)],
            lr=[pltpu.VMEM((K,), jnp.int32) for _ in range(NRB)],
            accum=pltpu.VMEM_SHARED((nhalf + APAD, D), jnp.float32),
            gsem=[pltpu.SemaphoreType.DMA for _ in range(NRB)],
            ssem=[pltpu.SemaphoreType.DMA for _ in range(NRB)],
        ),
    )


def _spmm_pair(edA, xA, edB, xB, nout, skipB=False):
    idxA, vA = edA
    idxB, vB = edB
    k = _make_pair(idxA.shape[1], idxB.shape[1], nout, skipB)
    return k(idxA, vA, xA, idxB, vB, xB)


def _mm_body(x_ref, w_ref, b_ref, o_ref):
    o_ref[...] = jnp.dot(x_ref[...], w_ref[...],
                         preferred_element_type=jnp.float32,
                         precision=lax.Precision.HIGHEST) + b_ref[...]


def _proj(x, w, b):
    n = x.shape[0]
    bm = 1024 if n % 1024 == 0 else 512
    return pl.pallas_call(
        _mm_body,
        grid=(n // bm,),
        in_specs=[pl.BlockSpec((bm, D), lambda i: (i, 0)),
                  pl.BlockSpec((D, D), lambda i: (0, 0)),
                  pl.BlockSpec((1, D), lambda i: (0, 0))],
        out_specs=pl.BlockSpec((bm, D), lambda i: (i, 0)),
        out_shape=jax.ShapeDtypeStruct((n, D), jnp.float32),
    )(x, w, b.reshape(1, D))


def _mean3_body(a_ref, b_ref, c_ref, o_ref):
    o_ref[...] = (a_ref[...] + b_ref[...] + c_ref[...]) * (1.0 / 3.0)


def _mean3(a, b, c):
    n = a.shape[0]
    bm = 1024 if n % 1024 == 0 else 512
    spec = pl.BlockSpec((bm, D), lambda i: (i, 0))
    return pl.pallas_call(
        _mean3_body,
        grid=(n // bm,),
        in_specs=[spec, spec, spec],
        out_specs=spec,
        out_shape=jax.ShapeDtypeStruct((n, D), jnp.float32),
    )(a, b, c)


def _prep_edges(idx, val):
    e = idx.shape[1]
    epad = -(-e // EDGE_ALIGN) * EDGE_ALIGN
    return jnp.pad(idx, ((0, 0), (0, epad - e))), jnp.pad(val, (0, epad - e))


def kernel(hg_ul_idx, hg_ul_val, hg_ut_idx, hg_ut_val, hg_ua_idx, hg_ua_val,
           hg_l_idx, hg_l_val, hg_t_idx, hg_t_val, hg_a_idx, hg_a_val,
           vtoe_lu_idx, vtoe_lu_val, vtoe_tu_idx, vtoe_tu_val,
           vtoe_au_idx, vtoe_au_val, vtoe_ul_idx, vtoe_ul_val,
           vtoe_ut_idx, vtoe_ut_val, vtoe_ua_idx, vtoe_ua_val,
           user_emb, loc_emb, time_emb, act_emb, W_l, b_l, W_t, b_t, W_a, b_a):
    U, L = user_emb.shape[0], loc_emb.shape[0]

    hg_ul = _prep_edges(hg_ul_idx, hg_ul_val)
    hg_ut = _prep_edges(hg_ut_idx, hg_ut_val)
    hg_ua = _prep_edges(hg_ua_idx, hg_ua_val)
    hg_l = _prep_edges(hg_l_idx, hg_l_val)
    hg_t = _prep_edges(hg_t_idx, hg_t_val)
    hg_a = _prep_edges(hg_a_idx, hg_a_val)
    vt_lu = _prep_edges(vtoe_lu_idx, vtoe_lu_val)
    vt_tu = _prep_edges(vtoe_tu_idx, vtoe_tu_val)
    vt_au = _prep_edges(vtoe_au_idx, vtoe_au_val)
    vt_ul = _prep_edges(vtoe_ul_idx, vtoe_ul_val)
    vt_ut = _prep_edges(vtoe_ut_idx, vtoe_ut_val)
    vt_ua = _prep_edges(vtoe_ua_idx, vtoe_ua_val)

    pad_rows = lambda x: jnp.pad(x, ((0, NPAD_BIG - x.shape[0]), (0, 0)))
    ue = pad_rows(user_emb)
    u_l = _proj(ue, W_l, b_l)
    u_t = _proj(ue, W_t, b_t)
    u_a = _proj(ue, W_a, b_a)
    l = pad_rows(loc_emb)
    t, a = time_emb, act_emb

    u_l_all, u_t_all, u_a_all = [u_l], [u_t], [u_a]
    l_all, t_all, a_all = [l], [t], [a]
    for _ in range(2):
        u_l1 = _spmm_pair(hg_ul, u_l, vt_lu, l, NPAD_BIG)
        l1 = _spmm_pair(hg_l, l, vt_ul, u_l, NPAD_BIG)
        u_t1 = _spmm_pair(hg_ut, u_t, vt_tu, t, NPAD_BIG, skipB=True)
        t1 = _spmm_pair(hg_t, t, vt_ut, u_t, NPAD_SMALL)
        u_a1 = _spmm_pair(hg_ua, u_a, vt_au, a, NPAD_BIG, skipB=True)
        a1 = _spmm_pair(hg_a, a, vt_ua, u_a, NPAD_SMALL)
        u_l, l, u_t, t, u_a, a = u_l1, l1, u_t1, t1, u_a1, a1
        u_l_all.append(u_l); u_t_all.append(u_t); u_a_all.append(u_a)
        l_all.append(l); t_all.append(t); a_all.append(a)

    u_l_f = _mean3(*u_l_all)[:U]
    u_t_f = _mean3(*u_t_all)[:U]
    u_a_f = _mean3(*u_a_all)[:U]
    l_f = _mean3(*l_all)[:L]
    t_f = _mean3(*t_all)
    a_f = _mean3(*a_all)
    return (u_l_f, u_t_f, u_a_f, l_f, t_f, a_f)


# speedup vs baseline: 2.0524x; 1.0357x over previous
import functools

import jax
import jax.numpy as jnp
from jax import lax
from jax.experimental import pallas as pl
from jax.experimental.pallas import tpu as pltpu
from jax.experimental.pallas import tpu_sc as plsc

D = 64
K = 96
SB = 6
NRB = 2
SBE = SB * K
NS = 16
NC = 2
EDGE_ALIGN = NS * K * SB
NPAD_BIG = 50176
NPAD_SMALL = 8192
APAD = 256

_GDN = lax.GatherDimensionNumbers(
    offset_dims=(), collapsed_slice_dims=(0,), start_index_map=(0,))


def _lane_bcast(v16, e):
    idx = jnp.full((16, 1), e, jnp.int32)
    return lax.gather(v16, idx, _GDN, slice_sizes=(1,),
                      mode=lax.GatherScatterMode.PROMISE_IN_BOUNDS)


def _pair_body(EA, EB, NOUT, skipB, rows_max_b,
               idxA, valA, xA, idxB, valB, xB, out,
               idx_st, val_st, rb, ob, lr, accum, gsem, ssem):
    nhalf = NOUT // 2
    cid = lax.axis_index("c")
    sid = lax.axis_index("s")
    base = cid * nhalf

    zeros16 = jnp.zeros((16,), jnp.float32)

    def zrow(i, _):
        for j in range(4):
            rb[0][i, pl.ds(j * 16, 16)] = zeros16
        return 0

    lax.fori_loop(0, K, zrow, 0)
    acc_rows_per_sub = (nhalf + APAD) // NS
    z0 = sid * acc_rows_per_sub
    zfull, zrem = acc_rows_per_sub // K, acc_rows_per_sub % K
    for i in range(zfull):
        pltpu.sync_copy(rb[0], accum.at[pl.ds(z0 + i * K, K)])
    if zrem:
        pltpu.sync_copy(rb[0].at[pl.ds(0, zrem)],
                        accum.at[pl.ds(z0 + zfull * K, zrem)])
    plsc.subcore_barrier()

    def process(idx_hbm, val_hbm, x_hbm, nchunks):
        nsb = nchunks // SB
        e0 = sid * (nchunks * K)

        def stage(s):
            off = e0 + s * SBE
            pltpu.sync_copy(idx_hbm.at[:, pl.ds(off, SBE)], idx_st)
            pltpu.sync_copy(val_hbm.at[pl.ds(off, SBE)], val_st)

        def g_start(cc):
            b = cc % NRB
            pltpu.async_copy(x_hbm.at[idx_st.at[1, pl.ds(cc * K, K)]],
                             rb[b], gsem[b])

        def g_wait(cc):
            b = cc % NRB
            pltpu.make_async_copy(x_hbm.at[idx_st.at[1, pl.ds(cc * K, K)]],
                                  rb[b], gsem[b]).wait()

        def s_start(cc):
            b = cc % NRB
            pltpu.async_copy(ob[b], accum.at[lr[b]], ssem[b], add=True)

        def s_wait(b):
            pltpu.make_async_copy(ob[b], accum.at[lr[b]], ssem[b]).wait()

        def compute(cc):
            b = cc % NRB
            rbuf, obuf, lrow = rb[b], ob[b], lr[b]

            def group(q, _):
                coff = cc * K + q * 16
                r16 = idx_st[0, pl.ds(coff, 16)]
                lrl = r16 - base
                ok = (lrl >= 0) & (lrl < nhalf)
                lrow[pl.ds(q * 16, 16)] = jnp.where(ok, lrl, nhalf + sid)
                v16 = val_st[pl.ds(coff, 16)]
                for e in range(16):
                    vb = _lane_bcast(v16, e)
                    ei = q * 16 + e
                    for j in range(4):
                        obuf[ei, pl.ds(j * 16, 16)] = (
                            rbuf[ei, pl.ds(j * 16, 16)] * vb)
                return 0

            lax.fori_loop(0, K // 16, group, 0)

        def run_sb(first):
            g_start(0)
            for cc in range(SB):
                if cc + 1 < SB:
                    g_start(cc + 1)
                g_wait(cc)
                if (not first) or cc >= 2:
                    s_wait(cc % NRB)
                compute(cc)
                s_start(cc)

        stage(0)
        run_sb(True)

        def sb_body(s, _):
            stage(s)
            run_sb(False)
            return 0

        if nsb > 1:
            lax.fori_loop(1, nsb, sb_body, 0)
        for b in range(NRB):
            s_wait(b)

    process(idxA, valA, xA, EA // (NS * K))
    if skipB:

        @pl.when(cid == 0)
        def _():
            process(idxB, valB, xB, EB // (NS * K))
    else:
        process(idxB, valB, xB, EB // (NS * K))
    plsc.subcore_barrier()

    wb_rows = nhalf // NS
    w0 = sid * wb_rows
    wfull, wrem = wb_rows // K, wb_rows % K
    for i in range(wfull):
        pltpu.sync_copy(accum.at[pl.ds(w0 + i * K, K)],
                        out.at[pl.ds(base + w0 + i * K, K)])
    if wrem:
        pltpu.sync_copy(accum.at[pl.ds(w0 + wfull * K, wrem)],
                        out.at[pl.ds(base + w0 + wfull * K, wrem)])


@functools.lru_cache(maxsize=None)
def _make_pair(EA, EB, NOUT, skipB):
    mesh = plsc.VectorSubcoreMesh(core_axis_name="c", subcore_axis_name="s")
    nhalf = NOUT // 2
    return pl.kernel(
        functools.partial(_pair_body, EA, EB, NOUT, skipB, None),
        out_type=jax.ShapeDtypeStruct((NOUT, D), jnp.float32),
        mesh=mesh,
        compiler_params=pltpu.CompilerParams(use_tc_tiling_on_sc=False,
                                             needs_layout_passes=False),
        scratch_types=dict(
            idx_st=pltpu.VMEM((2, SBE), jnp.int32),
            val_st=pltpu.VMEM((SBE,), jnp.float32),
            rb=[pltpu.VMEM((K, D), jnp.float32) for _ in range(NRB)],
            ob=[pltpu.VMEM((K, D), jnp.float32) for _ in range(NRB)],
            lr=[pltpu.VMEM((K,), jnp.int32) for _ in range(NRB)],
            accum=pltpu.VMEM_SHARED((nhalf + APAD, D), jnp.float32),
            gsem=[pltpu.SemaphoreType.DMA for _ in range(NRB)],
            ssem=[pltpu.SemaphoreType.DMA for _ in range(NRB)],
        ),
    )


def _spmm_pair(edA, xA, edB, xB, nout, skipB=False):
    idxA, vA = edA
    idxB, vB = edB
    k = _make_pair(idxA.shape[1], idxB.shape[1], nout, skipB)
    return k(idxA, vA, xA, idxB, vB, xB)


def _mm_body(x_ref, w_ref, b_ref, o_ref):
    o_ref[...] = jnp.dot(x_ref[...], w_ref[...],
                         preferred_element_type=jnp.float32,
                         precision=lax.Precision.HIGHEST) + b_ref[...]


def _proj(x, w, b):
    n = x.shape[0]
    bm = 1024 if n % 1024 == 0 else 512
    return pl.pallas_call(
        _mm_body,
        grid=(n // bm,),
        in_specs=[pl.BlockSpec((bm, D), lambda i: (i, 0)),
                  pl.BlockSpec((D, D), lambda i: (0, 0)),
                  pl.BlockSpec((1, D), lambda i: (0, 0))],
        out_specs=pl.BlockSpec((bm, D), lambda i: (i, 0)),
        out_shape=jax.ShapeDtypeStruct((n, D), jnp.float32),
    )(x, w, b.reshape(1, D))


def _mean3_body(a_ref, b_ref, c_ref, o_ref):
    o_ref[...] = (a_ref[...] + b_ref[...] + c_ref[...]) * (1.0 / 3.0)


def _mean3(a, b, c):
    n = a.shape[0]
    bm = 1024 if n % 1024 == 0 else 512
    spec = pl.BlockSpec((bm, D), lambda i: (i, 0))
    return pl.pallas_call(
        _mean3_body,
        grid=(n // bm,),
        in_specs=[spec, spec, spec],
        out_specs=spec,
        out_shape=jax.ShapeDtypeStruct((n, D), jnp.float32),
    )(a, b, c)


def _prep_edges(idx, val):
    e = idx.shape[1]
    epad = -(-e // EDGE_ALIGN) * EDGE_ALIGN
    return jnp.pad(idx, ((0, 0), (0, epad - e))), jnp.pad(val, (0, epad - e))


def kernel(hg_ul_idx, hg_ul_val, hg_ut_idx, hg_ut_val, hg_ua_idx, hg_ua_val,
           hg_l_idx, hg_l_val, hg_t_idx, hg_t_val, hg_a_idx, hg_a_val,
           vtoe_lu_idx, vtoe_lu_val, vtoe_tu_idx, vtoe_tu_val,
           vtoe_au_idx, vtoe_au_val, vtoe_ul_idx, vtoe_ul_val,
           vtoe_ut_idx, vtoe_ut_val, vtoe_ua_idx, vtoe_ua_val,
           user_emb, loc_emb, time_emb, act_emb, W_l, b_l, W_t, b_t, W_a, b_a):
    U, L = user_emb.shape[0], loc_emb.shape[0]

    hg_ul = _prep_edges(hg_ul_idx, hg_ul_val)
    hg_ut = _prep_edges(hg_ut_idx, hg_ut_val)
    hg_ua = _prep_edges(hg_ua_idx, hg_ua_val)
    hg_l = _prep_edges(hg_l_idx, hg_l_val)
    hg_t = _prep_edges(hg_t_idx, hg_t_val)
    hg_a = _prep_edges(hg_a_idx, hg_a_val)
    vt_lu = _prep_edges(vtoe_lu_idx, vtoe_lu_val)
    vt_tu = _prep_edges(vtoe_tu_idx, vtoe_tu_val)
    vt_au = _prep_edges(vtoe_au_idx, vtoe_au_val)
    vt_ul = _prep_edges(vtoe_ul_idx, vtoe_ul_val)
    vt_ut = _prep_edges(vtoe_ut_idx, vtoe_ut_val)
    vt_ua = _prep_edges(vtoe_ua_idx, vtoe_ua_val)

    pad_rows = lambda x: jnp.pad(x, ((0, NPAD_BIG - x.shape[0]), (0, 0)))
    ue = pad_rows(user_emb)
    u_l = _proj(ue, W_l, b_l)
    u_t = _proj(ue, W_t, b_t)
    u_a = _proj(ue, W_a, b_a)
    l = pad_rows(loc_emb)
    t, a = time_emb, act_emb

    u_l_all, u_t_all, u_a_all = [u_l], [u_t], [u_a]
    l_all, t_all, a_all = [l], [t], [a]
    for _ in range(2):
        u_l1 = _spmm_pair(hg_ul, u_l, vt_lu, l, NPAD_BIG)
        l1 = _spmm_pair(hg_l, l, vt_ul, u_l, NPAD_BIG)
        u_t1 = _spmm_pair(hg_ut, u_t, vt_tu, t, NPAD_BIG, skipB=True)
        t1 = _spmm_pair(hg_t, t, vt_ut, u_t, NPAD_SMALL)
        u_a1 = _spmm_pair(hg_ua, u_a, vt_au, a, NPAD_BIG, skipB=True)
        a1 = _spmm_pair(hg_a, a, vt_ua, u_a, NPAD_SMALL)
        u_l, l, u_t, t, u_a, a = u_l1, l1, u_t1, t1, u_a1, a1
        u_l_all.append(u_l); u_t_all.append(u_t); u_a_all.append(u_a)
        l_all.append(l); t_all.append(t); a_all.append(a)

    u_l_f = _mean3(*u_l_all)[:U]
    u_t_f = _mean3(*u_t_all)[:U]
    u_a_f = _mean3(*u_a_all)[:U]
    l_f = _mean3(*l_all)[:L]
    t_f = _mean3(*t_all)
    a_f = _mean3(*a_all)
    return (u_l_f, u_t_f, u_a_f, l_f, t_f, a_f)
